# baseline (speedup 1.0000x reference)
import functools
import numpy as np
import jax
import jax.numpy as jnp
from jax import lax
from jax.experimental import pallas as pl
from jax.experimental.pallas import tpu as pltpu
from jax.experimental.pallas import tpu_sc as plsc

EPS = 1e-8
B, C, T, H, W = 16, 3, 32, 112, 112
NPIX = H * W
TOPK = int(0.1 * H * W)
LANES = 128
ONE_BITS = 0x3F800000


def _gauss1d_np(ks, sigma):
    x = np.arange(ks, dtype=np.float32) - (ks - 1) / 2.0
    g = np.exp(-0.5 * (x / sigma) ** 2).astype(np.float32)
    return g / g.sum()


def _blur_matrix():
    k1 = _gauss1d_np(15, 5.0).astype(np.float64)
    t = np.arange(-7, H + 7)
    r = np.where(t < 0, -t, np.where(t > H - 1, 2 * (H - 1) - t, t))
    M = np.zeros((H, H), np.float64)
    for a in range(15):
        for i in range(H):
            M[i, r[i + a]] += k1[a]
    return M.astype(np.float32)


def _window_np():
    ky = _gauss1d_np(H, H / 3.0)
    kx = _gauss1d_np(W, W / 3.0)
    k = np.outer(ky, kx)
    return (k / k.max()).astype(np.float32)


_BLUR_M = _blur_matrix()
_WIN = _window_np()


def _reflect_pad_matrix():
    t = np.arange(-7, H + 7)
    r = np.where(t < 0, -t, np.where(t > H - 1, 2 * (H - 1) - t, t))
    P = np.zeros((H + 14, H), np.float32)
    P[np.arange(H + 14), r] = 1.0
    return P


def _band_placement():
    E = np.zeros((15, H, H + 14), np.float32)
    for a in range(15):
        E[a, np.arange(H), np.arange(H) + a] = 1.0
    return E


_PAD_P = _reflect_pad_matrix()
_BAND_E = _band_placement()


def _stats_kernel(v_ref, diff_ref, mean_ref):
    v = v_ref[0]
    acc = None
    for c in range(C):
        xc = v[c * T:(c + 1) * T]
        d = jnp.abs(xc[:-1] - xc[1:])
        acc = d if acc is None else acc + d
    diff_ref[0] = acc.mean(axis=0)
    for c in range(C):
        mean_ref[0, c] = v[c * T + 1:(c + 1) * T].mean(axis=0)


def _dot(a, b):
    return jnp.dot(a, b, preferred_element_type=jnp.float32,
                   precision=jax.lax.Precision.HIGHEST)


def _mid_kernel(diff_ref, rgb_ref, A_ref, P_ref, win_ref,
                cmap_ref, fg_ref, bg_ref):
    xbf = diff_ref[0].astype(jnp.bfloat16).astype(jnp.float32)
    P = P_ref[...]
    xp = _dot(_dot(P, xbf), P.T)
    acc = jnp.zeros((H, W), jnp.float32)
    for b in range(15):
        Tb = _dot(A_ref[b], xp)
        acc = acc + Tb[:, b:b + W]
    y = acc
    y = y - jnp.min(y)
    y = y / (jnp.max(y) + EPS)
    mask = y * win_ref[...]

    r = rgb_ref[0, 0]
    g = rgb_ref[0, 1]
    bl = rgb_ref[0, 2]
    maxc = jnp.maximum(jnp.maximum(r, g), bl)
    minc = jnp.minimum(jnp.minimum(r, g), bl)
    vch = maxc
    deltac = maxc - minc
    s = deltac / (maxc + EPS)
    dsafe = jnp.where(deltac == 0.0, 1.0, deltac)
    rc = (maxc - r) / dsafe
    gc = (maxc - g) / dsafe
    bc = (maxc - bl) / dsafe
    h = jnp.where(maxc == r, bc - gc,
                  jnp.where(maxc == g, 2.0 + rc - bc, 4.0 + gc - rc))
    h = (h / 6.0) % 1.0
    h = jnp.where(deltac == 0.0, 0.0, h)

    hx = (s * jnp.cos(h * (2 * np.pi)) + 1.0) / 2.0
    hy = (s * jnp.sin(h * (2 * np.pi)) + 1.0) / 2.0
    hq = jnp.round(hx * 4.0 + 1.0)
    sq = jnp.round(hy * 4.0 + 1.0)
    vq = jnp.round(vch * 4.0 + 1.0)
    cmap = hq + (sq - 1.0) * 5.0 + (vq - 1.0) * 25.0

    mbits = jax.lax.bitcast_convert_type(mask, jnp.int32)
    idx = (jax.lax.broadcasted_iota(jnp.int32, (H, W), 0) * W
           + jax.lax.broadcasted_iota(jnp.int32, (H, W), 1))

    def thr_body(_, carry):
        flo, fhi, blo, bhi = carry
        fmid = flo + (fhi - flo) // 2
        bmid = (blo + bhi) // 2
        fok = jnp.sum((mbits >= fmid).astype(jnp.int32)) >= TOPK
        bok = jnp.sum((mbits <= bmid).astype(jnp.int32)) >= TOPK
        return (jnp.where(fok, fmid, flo), jnp.where(fok, fhi, fmid),
                jnp.where(bok, blo, bmid + 1), jnp.where(bok, bmid, bhi))

    tfg, _, _, tbg = jax.lax.fori_loop(
        0, 31, thr_body,
        (jnp.int32(0), jnp.int32(ONE_BITS + 1),
         jnp.int32(0), jnp.int32(ONE_BITS)))

    m_fg = TOPK - jnp.sum((mbits > tfg).astype(jnp.int32))
    tie_fg = mbits == tfg
    m_bg = TOPK - jnp.sum((mbits < tbg).astype(jnp.int32))
    tie_bg = mbits == tbg

    def cut_body(_, carry):
        flo, fhi, blo, bhi = carry
        fmid = (flo + fhi) // 2
        bmid = (blo + bhi) // 2
        fok = jnp.sum((tie_fg & (idx < fmid)).astype(jnp.int32)) >= m_fg
        bok = jnp.sum((tie_bg & (idx < bmid)).astype(jnp.int32)) >= m_bg
        return (jnp.where(fok, flo, fmid + 1), jnp.where(fok, fmid, fhi),
                jnp.where(bok, blo, bmid + 1), jnp.where(bok, bmid, bhi))

    _, cfg, _, cbg = jax.lax.fori_loop(
        0, 15, cut_body,
        (jnp.int32(0), jnp.int32(NPIX + 1),
         jnp.int32(0), jnp.int32(NPIX + 1)))

    fgf = ((mbits > tfg) | (tie_fg & (idx < cfg))).astype(jnp.float32)
    bgf = ((mbits < tbg) | (tie_bg & (idx < cbg))).astype(jnp.float32)

    cmap_ref[0] = cmap.astype(jnp.int32)
    fg_ref[0] = fgf
    bg_ref[0] = bgf


def _sc_seg(cmap_i, fgf, bgf):
    mesh = plsc.VectorSubcoreMesh(core_axis_name="c", subcore_axis_name="s")

    @functools.partial(
        pl.kernel, mesh=mesh,
        out_type=jax.ShapeDtypeStruct((B, NPIX), jnp.float32),
        compiler_params=pltpu.CompilerParams(needs_layout_passes=False),
        scratch_types=[
            pltpu.VMEM((NPIX,), jnp.int32),
            pltpu.VMEM((NPIX,), jnp.float32),
            pltpu.VMEM((NPIX,), jnp.float32),
            pltpu.VMEM((NPIX,), jnp.float32),
            pltpu.VMEM((16 * 128,), jnp.float32),
            pltpu.VMEM((16 * 128,), jnp.float32),
            pltpu.VMEM((128,), jnp.float32),
        ],
    )
    def k(cmap_hbm, fg_hbm, bg_hbm, out_hbm,
          cmap_v, fg_v, bg_v, pr_v, hfg_v, hbg_v, ratio_v):
        wid = lax.axis_index("s") * 2 + lax.axis_index("c")

        @pl.when(wid < B)
        def _():
            pltpu.sync_copy(cmap_hbm.at[wid], cmap_v)
            pltpu.sync_copy(fg_hbm.at[wid], fg_v)
            pltpu.sync_copy(bg_hbm.at[wid], bg_v)
            z16 = jnp.zeros((16,), jnp.float32)

            def zbody(j, _):
                hfg_v[pl.ds(j * 16, 16)] = z16
                hbg_v[pl.ds(j * 16, 16)] = z16
                return 0

            lax.fori_loop(0, 128, zbody, 0)

            lanes = lax.iota(jnp.int32, 16)

            def hbody(i, _):
                cm = cmap_v[pl.ds(i * 16, 16)]
                pidx = lanes * 128 + cm
                plsc.addupdate_scatter(hfg_v, [pidx], fg_v[pl.ds(i * 16, 16)])
                plsc.addupdate_scatter(hbg_v, [pidx], bg_v[pl.ds(i * 16, 16)])
                return 0

            lax.fori_loop(0, NPIX // 16, hbody, 0)

            sf_acc = jnp.zeros((16,), jnp.float32)
            sb_acc = jnp.zeros((16,), jnp.float32)
            for j in range(8):
                af = jnp.zeros((16,), jnp.float32)
                ab = jnp.zeros((16,), jnp.float32)
                for l in range(16):
                    af = af + hfg_v[pl.ds(l * 128 + 16 * j, 16)]
                    ab = ab + hbg_v[pl.ds(l * 128 + 16 * j, 16)]
                binv = lax.iota(jnp.int32, 16) + 16 * j
                valid = jnp.where(binv < 125, 1.0, 0.0)
                nf = af * valid
                nb = (ab + 1.0) * valid
                hfg_v[pl.ds(16 * j, 16)] = nf
                hbg_v[pl.ds(16 * j, 16)] = nb
                sf_acc = sf_acc + nf
                sb_acc = sb_acc + nb
            sfg = jnp.sum(sf_acc)
            sbg = jnp.sum(sb_acc)
            for j in range(8):
                nf = hfg_v[pl.ds(16 * j, 16)]
                nb = hbg_v[pl.ds(16 * j, 16)]
                df = nf / (sfg + EPS)
                db = nb / (sbg + EPS)
                binv = lax.iota(jnp.int32, 16) + 16 * j
                ratio_v[pl.ds(16 * j, 16)] = jnp.where(
                    binv < 125, df / (db + df), 0.0)

            def gbody(i, _):
                cm = cmap_v[pl.ds(i * 16, 16)]
                cmc = jnp.minimum(cm, 124)
                pr_v[pl.ds(i * 16, 16)] = plsc.load_gather(ratio_v, [cmc])
                return 0

            lax.fori_loop(0, NPIX // 16, gbody, 0)
            pltpu.sync_copy(pr_v, out_hbm.at[wid])

    return k(cmap_i, fgf, bgf)


def _fin_kernel(pr_ref, M_ref, out_ref):
    Mb = M_ref[...]
    pr = pr_ref[0]
    y2 = jnp.dot(jnp.dot(Mb, pr, preferred_element_type=jnp.float32, precision=jax.lax.Precision.HIGHEST), Mb.T,
                 preferred_element_type=jnp.float32, precision=jax.lax.Precision.HIGHEST)
    y2 = y2 - jnp.min(y2)
    y2 = y2 / (jnp.max(y2) + EPS)
    out_ref[0] = y2


def _compose_kernel(cur_ref, prev_ref, m_ref, out_ref):
    m = m_ref[0][None]
    out_ref[0] = prev_ref[0] * (1.0 - m) + cur_ref[0] * m


def kernel(video_clips):
    vf = video_clips.reshape(B, C * T, H, W)
    Mc = jnp.asarray(_BLUR_M)

    kv = jnp.arange(15, dtype=jnp.float32) - 7.0
    g1 = jnp.exp(-0.5 * (kv / 5.0) ** 2)
    g1 = g1 / g1.sum()
    k2bf = jnp.outer(g1, g1).astype(jnp.bfloat16).astype(jnp.float32)
    Amat = jnp.einsum('aij,ab->bij', jnp.asarray(_BAND_E), k2bf,
                      precision=jax.lax.Precision.HIGHEST)
    wv = jnp.arange(H, dtype=jnp.float32) - (H - 1) / 2.0
    gw = jnp.exp(-0.5 * (wv / (H / 3.0)) ** 2)
    gw = gw / gw.sum()
    win2 = jnp.outer(gw, gw)
    win = win2 / jnp.max(win2)
    Pmat = jnp.asarray(_PAD_P)

    diff, rgbmean = pl.pallas_call(
        _stats_kernel,
        grid=(B,),
        in_specs=[pl.BlockSpec((1, C * T, H, W), lambda b: (b, 0, 0, 0))],
        out_specs=[pl.BlockSpec((1, H, W), lambda b: (b, 0, 0)),
                   pl.BlockSpec((1, C, H, W), lambda b: (b, 0, 0, 0))],
        out_shape=[jax.ShapeDtypeStruct((B, H, W), jnp.float32),
                   jax.ShapeDtypeStruct((B, C, H, W), jnp.float32)],
        compiler_params=pltpu.CompilerParams(
            dimension_semantics=("parallel",)),
    )(vf)

    cmap_i, fgf, bgf = pl.pallas_call(
        _mid_kernel,
        grid=(B,),
        in_specs=[pl.BlockSpec((1, H, W), lambda b: (b, 0, 0)),
                  pl.BlockSpec((1, C, H, W), lambda b: (b, 0, 0, 0)),
                  pl.BlockSpec((15, H, H + 14), lambda b: (0, 0, 0)),
                  pl.BlockSpec((H + 14, H), lambda b: (0, 0)),
                  pl.BlockSpec((H, W), lambda b: (0, 0))],
        out_specs=[pl.BlockSpec((1, H, W), lambda b: (b, 0, 0)),
                   pl.BlockSpec((1, H, W), lambda b: (b, 0, 0)),
                   pl.BlockSpec((1, H, W), lambda b: (b, 0, 0))],
        out_shape=[jax.ShapeDtypeStruct((B, H, W), jnp.int32),
                   jax.ShapeDtypeStruct((B, H, W), jnp.float32),
                   jax.ShapeDtypeStruct((B, H, W), jnp.float32)],
        compiler_params=pltpu.CompilerParams(
            dimension_semantics=("parallel",)),
    )(diff, rgbmean, Amat, Pmat, win)

    pr = _sc_seg(cmap_i.reshape(B, NPIX), fgf.reshape(B, NPIX),
                 bgf.reshape(B, NPIX))

    mask2 = pl.pallas_call(
        _fin_kernel,
        grid=(B,),
        in_specs=[pl.BlockSpec((1, H, W), lambda b: (b, 0, 0)),
                  pl.BlockSpec((H, W), lambda b: (0, 0))],
        out_specs=pl.BlockSpec((1, H, W), lambda b: (b, 0, 0)),
        out_shape=jax.ShapeDtypeStruct((B, H, W), jnp.float32),
        compiler_params=pltpu.CompilerParams(
            dimension_semantics=("parallel",)),
    )(pr.reshape(B, H, W), Mc)

    out = pl.pallas_call(
        _compose_kernel,
        grid=(B,),
        in_specs=[pl.BlockSpec((1, C * T, H, W), lambda b: (b, 0, 0, 0)),
                  pl.BlockSpec((1, C * T, H, W),
                               lambda b: ((b + B - 1) % B, 0, 0, 0)),
                  pl.BlockSpec((1, H, W), lambda b: (b, 0, 0))],
        out_specs=pl.BlockSpec((1, C * T, H, W), lambda b: (b, 0, 0, 0)),
        out_shape=jax.ShapeDtypeStruct((B, C * T, H, W), jnp.float32),
        compiler_params=pltpu.CompilerParams(
            dimension_semantics=("parallel",)),
    )(vf, vf, mask2)

    return out.reshape(B, C, T, H, W)


# speedup vs baseline: 1.0706x; 1.0706x over previous
import functools
import numpy as np
import jax
import jax.numpy as jnp
from jax import lax
from jax.experimental import pallas as pl
from jax.experimental.pallas import tpu as pltpu
from jax.experimental.pallas import tpu_sc as plsc

EPS = 1e-8
B, C, T, H, W = 16, 3, 32, 112, 112
NPIX = H * W
TOPK = int(0.1 * H * W)
LANES = 128
ONE_BITS = 0x3F800000


def _gauss1d_np(ks, sigma):
    x = np.arange(ks, dtype=np.float32) - (ks - 1) / 2.0
    g = np.exp(-0.5 * (x / sigma) ** 2).astype(np.float32)
    return g / g.sum()


def _blur_matrix():
    k1 = _gauss1d_np(15, 5.0).astype(np.float64)
    t = np.arange(-7, H + 7)
    r = np.where(t < 0, -t, np.where(t > H - 1, 2 * (H - 1) - t, t))
    M = np.zeros((H, H), np.float64)
    for a in range(15):
        for i in range(H):
            M[i, r[i + a]] += k1[a]
    return M.astype(np.float32)


def _window_np():
    ky = _gauss1d_np(H, H / 3.0)
    kx = _gauss1d_np(W, W / 3.0)
    k = np.outer(ky, kx)
    return (k / k.max()).astype(np.float32)


_BLUR_M = _blur_matrix()
_WIN = _window_np()


def _reflect_pad_matrix():
    t = np.arange(-7, H + 7)
    r = np.where(t < 0, -t, np.where(t > H - 1, 2 * (H - 1) - t, t))
    P = np.zeros((H + 14, H), np.float32)
    P[np.arange(H + 14), r] = 1.0
    return P


def _band_placement():
    E = np.zeros((15, H, H + 14), np.float32)
    for a in range(15):
        E[a, np.arange(H), np.arange(H) + a] = 1.0
    return E


_PAD_P = _reflect_pad_matrix()
_BAND_E = _band_placement()


def _stats_kernel(v_ref, diff_ref, mean_ref):
    v = v_ref[0]
    acc = None
    for c in range(C):
        xc = v[c * T:(c + 1) * T]
        d = jnp.abs(xc[:-1] - xc[1:])
        acc = d if acc is None else acc + d
    diff_ref[0] = acc.mean(axis=0)
    for c in range(C):
        mean_ref[0, c] = v[c * T + 1:(c + 1) * T].mean(axis=0)


def _dot(a, b):
    return jnp.dot(a, b, preferred_element_type=jnp.float32,
                   precision=jax.lax.Precision.HIGHEST)


def _mid_kernel(diff_ref, rgb_ref, A_ref, P_ref, win_ref,
                cmap_ref, fg_ref, bg_ref):
    xbf = diff_ref[0].astype(jnp.bfloat16).astype(jnp.float32)
    P = P_ref[...]
    xp = _dot(_dot(P, xbf), P.T)
    acc = jnp.zeros((H, W), jnp.float32)
    for b in range(15):
        Tb = _dot(A_ref[b], xp)
        acc = acc + Tb[:, b:b + W]
    y = acc
    y = y - jnp.min(y)
    y = y / (jnp.max(y) + EPS)
    mask = y * win_ref[...]

    r = rgb_ref[0, 0]
    g = rgb_ref[0, 1]
    bl = rgb_ref[0, 2]
    maxc = jnp.maximum(jnp.maximum(r, g), bl)
    minc = jnp.minimum(jnp.minimum(r, g), bl)
    vch = maxc
    deltac = maxc - minc
    s = deltac / (maxc + EPS)
    dsafe = jnp.where(deltac == 0.0, 1.0, deltac)
    rc = (maxc - r) / dsafe
    gc = (maxc - g) / dsafe
    bc = (maxc - bl) / dsafe
    h = jnp.where(maxc == r, bc - gc,
                  jnp.where(maxc == g, 2.0 + rc - bc, 4.0 + gc - rc))
    h = (h / 6.0) % 1.0
    h = jnp.where(deltac == 0.0, 0.0, h)

    hx = (s * jnp.cos(h * (2 * np.pi)) + 1.0) / 2.0
    hy = (s * jnp.sin(h * (2 * np.pi)) + 1.0) / 2.0
    hq = jnp.round(hx * 4.0 + 1.0)
    sq = jnp.round(hy * 4.0 + 1.0)
    vq = jnp.round(vch * 4.0 + 1.0)
    cmap = hq + (sq - 1.0) * 5.0 + (vq - 1.0) * 25.0

    mbits = jax.lax.bitcast_convert_type(mask, jnp.int32)
    idx = (jax.lax.broadcasted_iota(jnp.int32, (H, W), 0) * W
           + jax.lax.broadcasted_iota(jnp.int32, (H, W), 1))

    def thr_body(_, carry):
        flo, fhi, blo, bhi = carry
        fmid = flo + (fhi - flo) // 2
        bmid = (blo + bhi) // 2
        fok = jnp.sum((mbits >= fmid).astype(jnp.int32)) >= TOPK
        bok = jnp.sum((mbits <= bmid).astype(jnp.int32)) >= TOPK
        return (jnp.where(fok, fmid, flo), jnp.where(fok, fhi, fmid),
                jnp.where(bok, blo, bmid + 1), jnp.where(bok, bmid, bhi))

    tfg, _, _, tbg = jax.lax.fori_loop(
        0, 31, thr_body,
        (jnp.int32(0), jnp.int32(ONE_BITS + 1),
         jnp.int32(0), jnp.int32(ONE_BITS)))

    m_fg = TOPK - jnp.sum((mbits > tfg).astype(jnp.int32))
    tie_fg = mbits == tfg
    m_bg = TOPK - jnp.sum((mbits < tbg).astype(jnp.int32))
    tie_bg = mbits == tbg

    def cut_body(_, carry):
        flo, fhi, blo, bhi = carry
        fmid = (flo + fhi) // 2
        bmid = (blo + bhi) // 2
        fok = jnp.sum((tie_fg & (idx < fmid)).astype(jnp.int32)) >= m_fg
        bok = jnp.sum((tie_bg & (idx < bmid)).astype(jnp.int32)) >= m_bg
        return (jnp.where(fok, flo, fmid + 1), jnp.where(fok, fmid, fhi),
                jnp.where(bok, blo, bmid + 1), jnp.where(bok, bmid, bhi))

    _, cfg, _, cbg = jax.lax.fori_loop(
        0, 15, cut_body,
        (jnp.int32(0), jnp.int32(NPIX + 1),
         jnp.int32(0), jnp.int32(NPIX + 1)))

    fgf = ((mbits > tfg) | (tie_fg & (idx < cfg))).astype(jnp.float32)
    bgf = ((mbits < tbg) | (tie_bg & (idx < cbg))).astype(jnp.float32)

    cmap_ref[0] = cmap.astype(jnp.int32)
    fg_ref[0] = fgf
    bg_ref[0] = bgf


def _sc_seg(cmap_i, fgf, bgf):
    mesh = plsc.VectorSubcoreMesh(core_axis_name="c", subcore_axis_name="s")

    @functools.partial(
        pl.kernel, mesh=mesh,
        out_type=jax.ShapeDtypeStruct((B, NPIX), jnp.float32),
        compiler_params=pltpu.CompilerParams(needs_layout_passes=False),
        scratch_types=[
            pltpu.VMEM((NPIX,), jnp.int32),
            pltpu.VMEM((NPIX,), jnp.float32),
            pltpu.VMEM((NPIX,), jnp.float32),
            pltpu.VMEM((NPIX,), jnp.float32),
            pltpu.VMEM((16 * 128,), jnp.float32),
            pltpu.VMEM((16 * 128,), jnp.float32),
            pltpu.VMEM((128,), jnp.float32),
        ],
    )
    def k(cmap_hbm, fg_hbm, bg_hbm, out_hbm,
          cmap_v, fg_v, bg_v, pr_v, hfg_v, hbg_v, ratio_v):
        wid = lax.axis_index("s") * 2 + lax.axis_index("c")

        @pl.when(wid < B)
        def _():
            pltpu.sync_copy(cmap_hbm.at[wid], cmap_v)
            pltpu.sync_copy(fg_hbm.at[wid], fg_v)
            pltpu.sync_copy(bg_hbm.at[wid], bg_v)
            z16 = jnp.zeros((16,), jnp.float32)

            def zbody(j, _):
                hfg_v[pl.ds(j * 16, 16)] = z16
                hbg_v[pl.ds(j * 16, 16)] = z16
                return 0

            lax.fori_loop(0, 128, zbody, 0)

            lanes = lax.iota(jnp.int32, 16)

            def hbody(i, _):
                cm = cmap_v[pl.ds(i * 16, 16)]
                pidx = lanes * 128 + cm
                plsc.addupdate_scatter(hfg_v, [pidx], fg_v[pl.ds(i * 16, 16)])
                plsc.addupdate_scatter(hbg_v, [pidx], bg_v[pl.ds(i * 16, 16)])
                return 0

            lax.fori_loop(0, NPIX // 16, hbody, 0)

            sf_acc = jnp.zeros((16,), jnp.float32)
            sb_acc = jnp.zeros((16,), jnp.float32)
            for j in range(8):
                af = jnp.zeros((16,), jnp.float32)
                ab = jnp.zeros((16,), jnp.float32)
                for l in range(16):
                    af = af + hfg_v[pl.ds(l * 128 + 16 * j, 16)]
                    ab = ab + hbg_v[pl.ds(l * 128 + 16 * j, 16)]
                binv = lax.iota(jnp.int32, 16) + 16 * j
                valid = jnp.where(binv < 125, 1.0, 0.0)
                nf = af * valid
                nb = (ab + 1.0) * valid
                hfg_v[pl.ds(16 * j, 16)] = nf
                hbg_v[pl.ds(16 * j, 16)] = nb
                sf_acc = sf_acc + nf
                sb_acc = sb_acc + nb
            sfg = jnp.sum(sf_acc)
            sbg = jnp.sum(sb_acc)
            for j in range(8):
                nf = hfg_v[pl.ds(16 * j, 16)]
                nb = hbg_v[pl.ds(16 * j, 16)]
                df = nf / (sfg + EPS)
                db = nb / (sbg + EPS)
                binv = lax.iota(jnp.int32, 16) + 16 * j
                ratio_v[pl.ds(16 * j, 16)] = jnp.where(
                    binv < 125, df / (db + df), 0.0)

            def gbody(i, _):
                cm = cmap_v[pl.ds(i * 16, 16)]
                cmc = jnp.minimum(cm, 124)
                pr_v[pl.ds(i * 16, 16)] = plsc.load_gather(ratio_v, [cmc])
                return 0

            lax.fori_loop(0, NPIX // 16, gbody, 0)
            pltpu.sync_copy(pr_v, out_hbm.at[wid])

    return k(cmap_i, fgf, bgf)


def _fin_kernel(pr_ref, M_ref, out_ref):
    Mb = M_ref[...]
    pr = pr_ref[0]
    y2 = jnp.dot(jnp.dot(Mb, pr, preferred_element_type=jnp.float32, precision=jax.lax.Precision.HIGHEST), Mb.T,
                 preferred_element_type=jnp.float32, precision=jax.lax.Precision.HIGHEST)
    y2 = y2 - jnp.min(y2)
    y2 = y2 / (jnp.max(y2) + EPS)
    out_ref[0] = y2


def _compose_kernel(cur_ref, m_ref, out_ref, prev_scr):
    i = pl.program_id(0)

    @pl.when(i > 0)
    def _():
        m = m_ref[0][None]
        out_ref[0] = prev_scr[...] * (1.0 - m) + cur_ref[0] * m

    prev_scr[...] = cur_ref[0]


def kernel(video_clips):
    vf = video_clips.reshape(B, C * T, H, W)
    Mc = jnp.asarray(_BLUR_M)

    kv = jnp.arange(15, dtype=jnp.float32) - 7.0
    g1 = jnp.exp(-0.5 * (kv / 5.0) ** 2)
    g1 = g1 / g1.sum()
    k2bf = jnp.outer(g1, g1).astype(jnp.bfloat16).astype(jnp.float32)
    Amat = jnp.einsum('aij,ab->bij', jnp.asarray(_BAND_E), k2bf,
                      precision=jax.lax.Precision.HIGHEST)
    wv = jnp.arange(H, dtype=jnp.float32) - (H - 1) / 2.0
    gw = jnp.exp(-0.5 * (wv / (H / 3.0)) ** 2)
    gw = gw / gw.sum()
    win2 = jnp.outer(gw, gw)
    win = win2 / jnp.max(win2)
    Pmat = jnp.asarray(_PAD_P)

    diff, rgbmean = pl.pallas_call(
        _stats_kernel,
        grid=(B,),
        in_specs=[pl.BlockSpec((1, C * T, H, W), lambda b: (b, 0, 0, 0))],
        out_specs=[pl.BlockSpec((1, H, W), lambda b: (b, 0, 0)),
                   pl.BlockSpec((1, C, H, W), lambda b: (b, 0, 0, 0))],
        out_shape=[jax.ShapeDtypeStruct((B, H, W), jnp.float32),
                   jax.ShapeDtypeStruct((B, C, H, W), jnp.float32)],
        compiler_params=pltpu.CompilerParams(
            dimension_semantics=("parallel",)),
    )(vf)

    cmap_i, fgf, bgf = pl.pallas_call(
        _mid_kernel,
        grid=(B,),
        in_specs=[pl.BlockSpec((1, H, W), lambda b: (b, 0, 0)),
                  pl.BlockSpec((1, C, H, W), lambda b: (b, 0, 0, 0)),
                  pl.BlockSpec((15, H, H + 14), lambda b: (0, 0, 0)),
                  pl.BlockSpec((H + 14, H), lambda b: (0, 0)),
                  pl.BlockSpec((H, W), lambda b: (0, 0))],
        out_specs=[pl.BlockSpec((1, H, W), lambda b: (b, 0, 0)),
                   pl.BlockSpec((1, H, W), lambda b: (b, 0, 0)),
                   pl.BlockSpec((1, H, W), lambda b: (b, 0, 0))],
        out_shape=[jax.ShapeDtypeStruct((B, H, W), jnp.int32),
                   jax.ShapeDtypeStruct((B, H, W), jnp.float32),
                   jax.ShapeDtypeStruct((B, H, W), jnp.float32)],
        compiler_params=pltpu.CompilerParams(
            dimension_semantics=("parallel",)),
    )(diff, rgbmean, Amat, Pmat, win)

    pr = _sc_seg(cmap_i.reshape(B, NPIX), fgf.reshape(B, NPIX),
                 bgf.reshape(B, NPIX))

    mask2 = pl.pallas_call(
        _fin_kernel,
        grid=(B,),
        in_specs=[pl.BlockSpec((1, H, W), lambda b: (b, 0, 0)),
                  pl.BlockSpec((H, W), lambda b: (0, 0))],
        out_specs=pl.BlockSpec((1, H, W), lambda b: (b, 0, 0)),
        out_shape=jax.ShapeDtypeStruct((B, H, W), jnp.float32),
        compiler_params=pltpu.CompilerParams(
            dimension_semantics=("parallel",)),
    )(pr.reshape(B, H, W), Mc)

    out = pl.pallas_call(
        _compose_kernel,
        grid=(B + 1,),
        in_specs=[pl.BlockSpec((1, C * T, H, W),
                               lambda i: ((i + B - 1) % B, 0, 0, 0)),
                  pl.BlockSpec((1, H, W),
                               lambda i: ((i + B - 1) % B, 0, 0))],
        out_specs=pl.BlockSpec((1, C * T, H, W),
                               lambda i: ((i + B - 1) % B, 0, 0, 0)),
        out_shape=jax.ShapeDtypeStruct((B, C * T, H, W), jnp.float32),
        scratch_shapes=[pltpu.VMEM((C * T, H, W), jnp.float32)],
        compiler_params=pltpu.CompilerParams(
            dimension_semantics=("arbitrary",)),
    )(vf, mask2)

    return out.reshape(B, C, T, H, W)


# speedup vs baseline: 1.1719x; 1.0946x over previous
import functools
import numpy as np
import jax
import jax.numpy as jnp
from jax import lax
from jax.experimental import pallas as pl
from jax.experimental.pallas import tpu as pltpu
from jax.experimental.pallas import tpu_sc as plsc

EPS = 1e-8
B, C, T, H, W = 16, 3, 32, 112, 112
NPIX = H * W
TOPK = int(0.1 * H * W)
LANES = 128
ONE_BITS = 0x3F800000


def _gauss1d_np(ks, sigma):
    x = np.arange(ks, dtype=np.float32) - (ks - 1) / 2.0
    g = np.exp(-0.5 * (x / sigma) ** 2).astype(np.float32)
    return g / g.sum()


def _blur_matrix():
    k1 = _gauss1d_np(15, 5.0).astype(np.float64)
    t = np.arange(-7, H + 7)
    r = np.where(t < 0, -t, np.where(t > H - 1, 2 * (H - 1) - t, t))
    M = np.zeros((H, H), np.float64)
    for a in range(15):
        for i in range(H):
            M[i, r[i + a]] += k1[a]
    return M.astype(np.float32)


def _window_np():
    ky = _gauss1d_np(H, H / 3.0)
    kx = _gauss1d_np(W, W / 3.0)
    k = np.outer(ky, kx)
    return (k / k.max()).astype(np.float32)


_BLUR_M = _blur_matrix()
_WIN = _window_np()


def _reflect_pad_matrix():
    t = np.arange(-7, H + 7)
    r = np.where(t < 0, -t, np.where(t > H - 1, 2 * (H - 1) - t, t))
    P = np.zeros((H + 14, H), np.float32)
    P[np.arange(H + 14), r] = 1.0
    return P


def _band_placement():
    E = np.zeros((15, H, H + 14), np.float32)
    for a in range(15):
        E[a, np.arange(H), np.arange(H) + a] = 1.0
    return E


_PAD_P = _reflect_pad_matrix()
_BAND_E = _band_placement()


def _dot(a, b):
    return jnp.dot(a, b, preferred_element_type=jnp.float32,
                   precision=jax.lax.Precision.HIGHEST)


def _mid_kernel(v_ref, A_ref, P_ref, win_ref,
                cmap_ref, fg_ref, bg_ref):
    v = v_ref[0]
    acc = None
    for c in range(C):
        xc = v[c * T:(c + 1) * T]
        d = jnp.abs(xc[:-1] - xc[1:])
        acc = d if acc is None else acc + d
    imdiff = acc.mean(axis=0)

    xbf = imdiff.astype(jnp.bfloat16).astype(jnp.float32)
    P = P_ref[...]
    xp = _dot(_dot(P, xbf), P.T)
    acc = jnp.zeros((H, W), jnp.float32)
    for b in range(15):
        Tb = _dot(A_ref[b], xp)
        acc = acc + Tb[:, b:b + W]
    y = acc
    y = y - jnp.min(y)
    y = y / (jnp.max(y) + EPS)
    mask = y * win_ref[...]

    r = v[1:T].mean(axis=0)
    g = v[T + 1:2 * T].mean(axis=0)
    bl = v[2 * T + 1:3 * T].mean(axis=0)
    maxc = jnp.maximum(jnp.maximum(r, g), bl)
    minc = jnp.minimum(jnp.minimum(r, g), bl)
    vch = maxc
    deltac = maxc - minc
    s = deltac / (maxc + EPS)
    dsafe = jnp.where(deltac == 0.0, 1.0, deltac)
    rc = (maxc - r) / dsafe
    gc = (maxc - g) / dsafe
    bc = (maxc - bl) / dsafe
    h = jnp.where(maxc == r, bc - gc,
                  jnp.where(maxc == g, 2.0 + rc - bc, 4.0 + gc - rc))
    h = (h / 6.0) % 1.0
    h = jnp.where(deltac == 0.0, 0.0, h)

    hx = (s * jnp.cos(h * (2 * np.pi)) + 1.0) / 2.0
    hy = (s * jnp.sin(h * (2 * np.pi)) + 1.0) / 2.0
    hq = jnp.round(hx * 4.0 + 1.0)
    sq = jnp.round(hy * 4.0 + 1.0)
    vq = jnp.round(vch * 4.0 + 1.0)
    cmap = hq + (sq - 1.0) * 5.0 + (vq - 1.0) * 25.0

    mbits = jax.lax.bitcast_convert_type(mask, jnp.int32)
    idx = (jax.lax.broadcasted_iota(jnp.int32, (H, W), 0) * W
           + jax.lax.broadcasted_iota(jnp.int32, (H, W), 1))

    def thr_body(_, carry):
        flo, fhi, blo, bhi = carry
        fmid = flo + (fhi - flo) // 2
        bmid = (blo + bhi) // 2
        fok = jnp.sum((mbits >= fmid).astype(jnp.int32)) >= TOPK
        bok = jnp.sum((mbits <= bmid).astype(jnp.int32)) >= TOPK
        return (jnp.where(fok, fmid, flo), jnp.where(fok, fhi, fmid),
                jnp.where(bok, blo, bmid + 1), jnp.where(bok, bmid, bhi))

    tfg, _, _, tbg = jax.lax.fori_loop(
        0, 31, thr_body,
        (jnp.int32(0), jnp.int32(ONE_BITS + 1),
         jnp.int32(0), jnp.int32(ONE_BITS)))

    m_fg = TOPK - jnp.sum((mbits > tfg).astype(jnp.int32))
    tie_fg = mbits == tfg
    m_bg = TOPK - jnp.sum((mbits < tbg).astype(jnp.int32))
    tie_bg = mbits == tbg

    def cut_body(_, carry):
        flo, fhi, blo, bhi = carry
        fmid = (flo + fhi) // 2
        bmid = (blo + bhi) // 2
        fok = jnp.sum((tie_fg & (idx < fmid)).astype(jnp.int32)) >= m_fg
        bok = jnp.sum((tie_bg & (idx < bmid)).astype(jnp.int32)) >= m_bg
        return (jnp.where(fok, flo, fmid + 1), jnp.where(fok, fmid, fhi),
                jnp.where(bok, blo, bmid + 1), jnp.where(bok, bmid, bhi))

    _, cfg, _, cbg = jax.lax.fori_loop(
        0, 15, cut_body,
        (jnp.int32(0), jnp.int32(NPIX + 1),
         jnp.int32(0), jnp.int32(NPIX + 1)))

    fgf = ((mbits > tfg) | (tie_fg & (idx < cfg))).astype(jnp.float32)
    bgf = ((mbits < tbg) | (tie_bg & (idx < cbg))).astype(jnp.float32)

    cmap_ref[0] = cmap.astype(jnp.int32)
    fg_ref[0] = fgf
    bg_ref[0] = bgf


def _sc_seg(cmap_i, fgf, bgf):
    mesh = plsc.VectorSubcoreMesh(core_axis_name="c", subcore_axis_name="s")

    @functools.partial(
        pl.kernel, mesh=mesh,
        out_type=jax.ShapeDtypeStruct((B, NPIX), jnp.float32),
        compiler_params=pltpu.CompilerParams(needs_layout_passes=False),
        scratch_types=[
            pltpu.VMEM((NPIX,), jnp.int32),
            pltpu.VMEM((NPIX,), jnp.float32),
            pltpu.VMEM((NPIX,), jnp.float32),
            pltpu.VMEM((NPIX,), jnp.float32),
            pltpu.VMEM((16 * 128,), jnp.float32),
            pltpu.VMEM((16 * 128,), jnp.float32),
            pltpu.VMEM((128,), jnp.float32),
        ],
    )
    def k(cmap_hbm, fg_hbm, bg_hbm, out_hbm,
          cmap_v, fg_v, bg_v, pr_v, hfg_v, hbg_v, ratio_v):
        wid = lax.axis_index("s") * 2 + lax.axis_index("c")

        @pl.when(wid < B)
        def _():
            pltpu.sync_copy(cmap_hbm.at[wid], cmap_v)
            pltpu.sync_copy(fg_hbm.at[wid], fg_v)
            pltpu.sync_copy(bg_hbm.at[wid], bg_v)
            z16 = jnp.zeros((16,), jnp.float32)

            def zbody(j, _):
                hfg_v[pl.ds(j * 16, 16)] = z16
                hbg_v[pl.ds(j * 16, 16)] = z16
                return 0

            lax.fori_loop(0, 128, zbody, 0)

            lanes = lax.iota(jnp.int32, 16)

            def hbody(i, _):
                cm = cmap_v[pl.ds(i * 16, 16)]
                pidx = lanes * 128 + cm
                plsc.addupdate_scatter(hfg_v, [pidx], fg_v[pl.ds(i * 16, 16)])
                plsc.addupdate_scatter(hbg_v, [pidx], bg_v[pl.ds(i * 16, 16)])
                return 0

            lax.fori_loop(0, NPIX // 16, hbody, 0)

            sf_acc = jnp.zeros((16,), jnp.float32)
            sb_acc = jnp.zeros((16,), jnp.float32)
            for j in range(8):
                af = jnp.zeros((16,), jnp.float32)
                ab = jnp.zeros((16,), jnp.float32)
                for l in range(16):
                    af = af + hfg_v[pl.ds(l * 128 + 16 * j, 16)]
                    ab = ab + hbg_v[pl.ds(l * 128 + 16 * j, 16)]
                binv = lax.iota(jnp.int32, 16) + 16 * j
                valid = jnp.where(binv < 125, 1.0, 0.0)
                nf = af * valid
                nb = (ab + 1.0) * valid
                hfg_v[pl.ds(16 * j, 16)] = nf
                hbg_v[pl.ds(16 * j, 16)] = nb
                sf_acc = sf_acc + nf
                sb_acc = sb_acc + nb
            sfg = jnp.sum(sf_acc)
            sbg = jnp.sum(sb_acc)
            for j in range(8):
                nf = hfg_v[pl.ds(16 * j, 16)]
                nb = hbg_v[pl.ds(16 * j, 16)]
                df = nf / (sfg + EPS)
                db = nb / (sbg + EPS)
                binv = lax.iota(jnp.int32, 16) + 16 * j
                ratio_v[pl.ds(16 * j, 16)] = jnp.where(
                    binv < 125, df / (db + df), 0.0)

            def gbody(i, _):
                cm = cmap_v[pl.ds(i * 16, 16)]
                cmc = jnp.minimum(cm, 124)
                pr_v[pl.ds(i * 16, 16)] = plsc.load_gather(ratio_v, [cmc])
                return 0

            lax.fori_loop(0, NPIX // 16, gbody, 0)
            pltpu.sync_copy(pr_v, out_hbm.at[wid])

    return k(cmap_i, fgf, bgf)


def _compose_kernel(cur_ref, pr_ref, M_ref, out_ref, prev_scr):
    i = pl.program_id(0)

    @pl.when(i > 0)
    def _():
        Mb = M_ref[...]
        y2 = _dot(_dot(Mb, pr_ref[0]), Mb.T)
        y2 = y2 - jnp.min(y2)
        y2 = y2 / (jnp.max(y2) + EPS)
        m = y2[None]
        out_ref[0] = prev_scr[...] * (1.0 - m) + cur_ref[0] * m

    prev_scr[...] = cur_ref[0]


def kernel(video_clips):
    vf = video_clips.reshape(B, C * T, H, W)
    Mc = jnp.asarray(_BLUR_M)

    kv = jnp.arange(15, dtype=jnp.float32) - 7.0
    g1 = jnp.exp(-0.5 * (kv / 5.0) ** 2)
    g1 = g1 / g1.sum()
    k2bf = jnp.outer(g1, g1).astype(jnp.bfloat16).astype(jnp.float32)
    Amat = jnp.einsum('aij,ab->bij', jnp.asarray(_BAND_E), k2bf,
                      precision=jax.lax.Precision.HIGHEST)
    wv = jnp.arange(H, dtype=jnp.float32) - (H - 1) / 2.0
    gw = jnp.exp(-0.5 * (wv / (H / 3.0)) ** 2)
    gw = gw / gw.sum()
    win2 = jnp.outer(gw, gw)
    win = win2 / jnp.max(win2)
    Pmat = jnp.asarray(_PAD_P)

    cmap_i, fgf, bgf = pl.pallas_call(
        _mid_kernel,
        grid=(B,),
        in_specs=[pl.BlockSpec((1, C * T, H, W), lambda b: (b, 0, 0, 0)),
                  pl.BlockSpec((15, H, H + 14), lambda b: (0, 0, 0)),
                  pl.BlockSpec((H + 14, H), lambda b: (0, 0)),
                  pl.BlockSpec((H, W), lambda b: (0, 0))],
        out_specs=[pl.BlockSpec((1, H, W), lambda b: (b, 0, 0)),
                   pl.BlockSpec((1, H, W), lambda b: (b, 0, 0)),
                   pl.BlockSpec((1, H, W), lambda b: (b, 0, 0))],
        out_shape=[jax.ShapeDtypeStruct((B, H, W), jnp.int32),
                   jax.ShapeDtypeStruct((B, H, W), jnp.float32),
                   jax.ShapeDtypeStruct((B, H, W), jnp.float32)],
        compiler_params=pltpu.CompilerParams(
            dimension_semantics=("parallel",)),
    )(vf, Amat, Pmat, win)

    pr = _sc_seg(cmap_i.reshape(B, NPIX), fgf.reshape(B, NPIX),
                 bgf.reshape(B, NPIX))

    out = pl.pallas_call(
        _compose_kernel,
        grid=(B + 1,),
        in_specs=[pl.BlockSpec((1, C * T, H, W),
                               lambda i: ((i + B - 1) % B, 0, 0, 0)),
                  pl.BlockSpec((1, H, W),
                               lambda i: ((i + B - 1) % B, 0, 0)),
                  pl.BlockSpec((H, W), lambda i: (0, 0))],
        out_specs=pl.BlockSpec((1, C * T, H, W),
                               lambda i: ((i + B - 1) % B, 0, 0, 0)),
        out_shape=jax.ShapeDtypeStruct((B, C * T, H, W), jnp.float32),
        scratch_shapes=[pltpu.VMEM((C * T, H, W), jnp.float32)],
        compiler_params=pltpu.CompilerParams(
            dimension_semantics=("arbitrary",)),
    )(vf, pr.reshape(B, H, W), Mc)

    return out.reshape(B, C, T, H, W)
